# baseline (speedup 1.0000x reference)
import jax
import jax.numpy as jnp
from jax import lax
from jax.experimental import pallas as pl
from jax.experimental.pallas import tpu as pltpu
from jax.experimental.pallas import tpu_sc as plsc

D = 128
B = 16384
NC = 2
NS = 16
L = 16
NW = NC * NS
B_PER_W = B // NW
CHUNK = 128
NCHUNK = B_PER_W // CHUNK
RB = 3


def _body(w_hbm, o_hbm, i_hbm, res_hbm, idx_v, rows_v, out_v,
          semg, semo, semw):
    wid = lax.axis_index("s") * NC + lax.axis_index("c")
    base = wid * B_PER_W
    pltpu.sync_copy(i_hbm.at[pl.ds(base, B_PER_W)], idx_v)

    gathers = []
    for j in range(RB):
        gathers.append(
            pltpu.async_copy(
                w_hbm.at[idx_v.at[pl.ds(j * CHUNK, CHUNK)]],
                rows_v.at[j], semg.at[j]))
    act = pltpu.async_copy(o_hbm.at[pl.ds(base, B_PER_W)], out_v, semo)

    writes = []
    for j in range(NCHUNK):
        gathers[j].wait()
        if j == 0:
            act.wait()

        @plsc.parallel_loop(0, CHUNK, unroll=1)
        def mul_row(r):
            for c in range(D // L):
                s = pl.ds(c * L, L)
                out_v[j * CHUNK + r, s] = (
                    rows_v[j % RB, r, s] * out_v[j * CHUNK + r, s])

        if j + RB < NCHUNK:
            gathers.append(
                pltpu.async_copy(
                    w_hbm.at[idx_v.at[pl.ds((j + RB) * CHUNK, CHUNK)]],
                    rows_v.at[(j + RB) % RB], semg.at[(j + RB) % RB]))
        writes.append(
            pltpu.async_copy(out_v.at[pl.ds(j * CHUNK, CHUNK)],
                             res_hbm.at[pl.ds(base + j * CHUNK, CHUNK)],
                             semw))
    for w in writes:
        w.wait()


def kernel(out, taget_adj, weight):
    idx = taget_adj.astype(jnp.int32)
    mesh = plsc.VectorSubcoreMesh(core_axis_name="c", subcore_axis_name="s")
    k = pl.kernel(
        _body,
        mesh=mesh,
        out_type=jax.ShapeDtypeStruct((B, D), jnp.float32),
        scratch_types=[
            pltpu.VMEM((B_PER_W,), jnp.int32),
            pltpu.VMEM((RB, CHUNK, D), jnp.float32),
            pltpu.VMEM((B_PER_W, D), jnp.float32),
            pltpu.SemaphoreType.DMA((RB,)),
            pltpu.SemaphoreType.DMA,
            pltpu.SemaphoreType.DMA,
        ],
    )
    return k(weight, out, idx)


# speedup vs baseline: 1.0637x; 1.0637x over previous
import jax
import jax.numpy as jnp
from jax import lax
from jax.experimental import pallas as pl
from jax.experimental.pallas import tpu as pltpu
from jax.experimental.pallas import tpu_sc as plsc

D = 128
B = 16384
NC = 2
NS = 16
L = 16
NW = NC * NS
B_PER_W = B // NW
CHUNK = 128
NCHUNK = B_PER_W // CHUNK
RB = 3


def _body(w_hbm, o_hbm, i_hbm, res_hbm, idx_v, rows_v, out_v,
          semg, semo, semw):
    wid = lax.axis_index("s") * NC + lax.axis_index("c")
    base = wid * B_PER_W
    pltpu.sync_copy(i_hbm.at[pl.ds(base, B_PER_W)], idx_v)

    gathers, outs = [], []
    for j in range(NCHUNK):
        if j < RB:
            gathers.append(
                pltpu.async_copy(
                    w_hbm.at[idx_v.at[pl.ds(j * CHUNK, CHUNK)]],
                    rows_v.at[j], semg.at[j]))
        outs.append(
            pltpu.async_copy(o_hbm.at[pl.ds(base + j * CHUNK, CHUNK)],
                             out_v.at[j], semo.at[j]))
    writes = []
    for j in range(NCHUNK):
        gathers[j].wait()
        outs[j].wait()

        @plsc.parallel_loop(0, CHUNK, unroll=1)
        def mul_row(r):
            for c in range(D // L):
                s = pl.ds(c * L, L)
                out_v[j, r, s] = rows_v[j % RB, r, s] * out_v[j, r, s]

        if j + RB < NCHUNK:
            gathers.append(
                pltpu.async_copy(
                    w_hbm.at[idx_v.at[pl.ds((j + RB) * CHUNK, CHUNK)]],
                    rows_v.at[(j + RB) % RB], semg.at[(j + RB) % RB]))
        writes.append(
            pltpu.async_copy(out_v.at[j],
                             res_hbm.at[pl.ds(base + j * CHUNK, CHUNK)],
                             semw))
    for w in writes:
        w.wait()


def kernel(out, taget_adj, weight):
    idx = taget_adj.astype(jnp.int32)
    mesh = plsc.VectorSubcoreMesh(core_axis_name="c", subcore_axis_name="s")
    k = pl.kernel(
        _body,
        mesh=mesh,
        out_type=jax.ShapeDtypeStruct((B, D), jnp.float32),
        scratch_types=[
            pltpu.VMEM((B_PER_W,), jnp.int32),
            pltpu.VMEM((RB, CHUNK, D), jnp.float32),
            pltpu.VMEM((NCHUNK, CHUNK, D), jnp.float32),
            pltpu.SemaphoreType.DMA((RB,)),
            pltpu.SemaphoreType.DMA((NCHUNK,)),
            pltpu.SemaphoreType.DMA,
        ],
    )
    return k(weight, out, idx)


# speedup vs baseline: 1.0683x; 1.0044x over previous
import jax
import jax.numpy as jnp
from jax import lax
from jax.experimental import pallas as pl
from jax.experimental.pallas import tpu as pltpu
from jax.experimental.pallas import tpu_sc as plsc

D = 128
B = 16384
NC = 2
NS = 16
L = 16
NW = NC * NS
B_PER_W = B // NW
CHUNK = 128
NCHUNK = B_PER_W // CHUNK
RB = 3


def _body(w_hbm, o_hbm, i_hbm, res_hbm, idx_v, rows_v, out_v,
          semg, semo, semw):
    wid = lax.axis_index("s") * NC + lax.axis_index("c")
    base = wid * B_PER_W
    pltpu.sync_copy(i_hbm.at[pl.ds(base, B_PER_W)], idx_v)

    gathers, outs = [], []
    for j in range(NCHUNK):
        if j < RB:
            gathers.append(
                pltpu.async_copy(
                    w_hbm.at[idx_v.at[pl.ds(j * CHUNK, CHUNK)]],
                    rows_v.at[j], semg.at[j]))
        outs.append(
            pltpu.async_copy(o_hbm.at[pl.ds(base + j * CHUNK, CHUNK)],
                             out_v.at[j], semo.at[j]))
    writes = []
    for j in range(NCHUNK):
        gathers[j].wait()
        outs[j].wait()

        halves = 2 if j == NCHUNK - 1 else 1
        step = CHUNK // halves
        for h in range(halves):

            @plsc.parallel_loop(h * step, (h + 1) * step, unroll=1)
            def mul_row(r):
                for c in range(D // L):
                    s = pl.ds(c * L, L)
                    out_v[j, r, s] = rows_v[j % RB, r, s] * out_v[j, r, s]

            writes.append(
                pltpu.async_copy(
                    out_v.at[j, pl.ds(h * step, step)],
                    res_hbm.at[pl.ds(base + j * CHUNK + h * step, step)],
                    semw))
        if j + RB < NCHUNK:
            gathers.append(
                pltpu.async_copy(
                    w_hbm.at[idx_v.at[pl.ds((j + RB) * CHUNK, CHUNK)]],
                    rows_v.at[(j + RB) % RB], semg.at[(j + RB) % RB]))
    for w in writes:
        w.wait()


def kernel(out, taget_adj, weight):
    idx = taget_adj.astype(jnp.int32)
    mesh = plsc.VectorSubcoreMesh(core_axis_name="c", subcore_axis_name="s")
    k = pl.kernel(
        _body,
        mesh=mesh,
        out_type=jax.ShapeDtypeStruct((B, D), jnp.float32),
        scratch_types=[
            pltpu.VMEM((B_PER_W,), jnp.int32),
            pltpu.VMEM((RB, CHUNK, D), jnp.float32),
            pltpu.VMEM((NCHUNK, CHUNK, D), jnp.float32),
            pltpu.SemaphoreType.DMA((RB,)),
            pltpu.SemaphoreType.DMA((NCHUNK,)),
            pltpu.SemaphoreType.DMA,
        ],
    )
    return k(weight, out, idx)
